# baseline (speedup 1.0000x reference)
import jax
import jax.numpy as jnp
from jax import lax
from jax.experimental import pallas as pl
from jax.experimental.pallas import tpu as pltpu
from jax.experimental.pallas import tpu_sc as plsc

NW = 32
CH = 128
L = 16
NBUF = 11


def _body(table, offs, out, off_v, out_v, sem0, sem1, *bufs):
    sets = []
    for s in range(2):
        grp = bufs[s * NBUF:(s + 1) * NBUF]
        sets.append((grp[0:4], grp[4:7], grp[7:11]))

    pw = out.shape[0] // (8 * NW)
    nchunk = pw // CH
    n_per_batch = 96 * 96 * 96
    half = table.shape[0] // 2

    wid = lax.axis_index("s") * 2 + lax.axis_index("c")
    pbase = wid * pw
    vbase = (pbase // (out.shape[0] // (8 * 2))) * n_per_batch

    pltpu.sync_copy(offs.at[pl.ds(pbase * 3, pw * 3)], off_v)

    roff = (0, 4608, 48, 4656)

    def phase_a(cbase, st):
        idx_b, f_b, _ = st
        lanes = lax.iota(jnp.int32, L)
        for i in range(CH // L):
            fo = (cbase + i * L + lanes) * 3
            yc = plsc.load_gather(off_v, [fo])
            xc = plsc.load_gather(off_v, [fo + 1])
            zc = plsc.load_gather(off_v, [fo + 2])
            xi = xc.astype(jnp.int32)
            yi = yc.astype(jnp.int32)
            zi = zc.astype(jnp.int32)
            sl = pl.ds(i * L, L)
            f_b[0][sl] = zc - zi.astype(jnp.float32)
            f_b[1][sl] = xc - xi.astype(jnp.float32)
            f_b[2][sl] = yc - yi.astype(jnp.float32)
            v = vbase + yi * 9216 + xi * 96 + zi
            h = (v >> 1) + (v & 1) * half
            for c in range(4):
                idx_b[c][sl] = h + roff[c]

    def fire(st, sem):
        idx_b, _, r_b = st
        for c in range(4):
            pltpu.async_copy(table.at[idx_b[c]], r_b[c], sem)

    def drain(st, sem):
        idx_b, _, r_b = st
        for c in range(4):
            pltpu.make_async_copy(table.at[idx_b[c]], r_b[c], sem).wait()

    def blend(cbase, st):
        _, f_b, r_b = st
        lanes = lax.iota(jnp.int32, L)
        sel = lanes >> 3
        colid = lanes & 7
        for j in range(CH // 2):
            rvec = 2 * j + sel
            fz = plsc.load_gather(f_b[0], [rvec])
            fx = plsc.load_gather(f_b[1], [rvec])
            fy = plsc.load_gather(f_b[2], [rvec])
            a0 = plsc.load_gather(r_b[0], [rvec, colid])
            a1 = plsc.load_gather(r_b[0], [rvec, colid + 8])
            b0 = plsc.load_gather(r_b[1], [rvec, colid])
            b1 = plsc.load_gather(r_b[1], [rvec, colid + 8])
            c0 = plsc.load_gather(r_b[2], [rvec, colid])
            c1 = plsc.load_gather(r_b[2], [rvec, colid + 8])
            d0 = plsc.load_gather(r_b[3], [rvec, colid])
            d1 = plsc.load_gather(r_b[3], [rvec, colid + 8])
            a = a0 + fz * (a1 - a0)
            b = b0 + fz * (b1 - b0)
            c = c0 + fz * (c1 - c0)
            d = d0 + fz * (d1 - d0)
            e = a + fx * (c - a)
            f = b + fx * (d - b)
            out_v[pl.ds(j * L, L)] = e + fy * (f - e)
        pltpu.sync_copy(out_v, out.at[pl.ds((pbase + cbase) * 8, CH * 8)])

    phase_a(0, sets[0])
    fire(sets[0], sem0)

    def pair(i, carry):
        cb0 = (2 * i) * CH
        cb1 = cb0 + CH
        cb2 = cb0 + 2 * CH
        phase_a(cb1, sets[1])
        fire(sets[1], sem1)
        drain(sets[0], sem0)
        blend(cb0, sets[0])

        @pl.when(cb2 < pw)
        def _():
            phase_a(cb2, sets[0])
            fire(sets[0], sem0)

        drain(sets[1], sem1)
        blend(cb1, sets[1])
        return carry

    lax.fori_loop(0, nchunk // 2, pair, 0)


def kernel(im, offsets):
    B, H, W, D, C = im.shape
    N = offsets.shape[1]
    nvox = B * H * W * D
    flat = im.reshape(nvox * C)
    table = jnp.concatenate(
        [flat, flat[C:], jnp.zeros((C,), jnp.float32)]).reshape(nvox, 2 * C)
    offs = offsets.reshape(B * N * 3)

    mesh = plsc.VectorSubcoreMesh(core_axis_name="c", subcore_axis_name="s")
    pw = B * N // NW
    scratch = [
        pltpu.VMEM((pw * 3,), jnp.float32),
        pltpu.VMEM((CH * 8,), jnp.float32),
        pltpu.SemaphoreType.DMA,
        pltpu.SemaphoreType.DMA,
    ]
    for _ in range(2):
        scratch += [pltpu.VMEM((CH,), jnp.int32) for _ in range(4)]
        scratch += [pltpu.VMEM((CH,), jnp.float32) for _ in range(3)]
        scratch += [pltpu.VMEM((CH, 2 * C), jnp.float32) for _ in range(4)]

    run = pl.kernel(
        _body,
        out_type=jax.ShapeDtypeStruct((B * N * C,), jnp.float32),
        mesh=mesh,
        scratch_types=scratch,
        compiler_params=pltpu.CompilerParams(
            needs_layout_passes=False, use_tc_tiling_on_sc=False),
    )
    return run(table, offs).reshape(B, N, C)


# speedup vs baseline: 1.3614x; 1.3614x over previous
import jax
import jax.numpy as jnp
from jax import lax
from jax.experimental import pallas as pl
from jax.experimental.pallas import tpu as pltpu
from jax.experimental.pallas import tpu_sc as plsc

NW = 32
CH = 128
L = 16
NBUF = 19


def _body(table, offs, out, off_v, out_v, sem0, sem1, *bufs):
    npts = offs.shape[0]

    sets = []
    for s in range(2):
        grp = bufs[s * NBUF:(s + 1) * NBUF]
        sets.append((grp[0:8], grp[8:11], grp[11:19]))

    pw = npts // NW
    wid = lax.axis_index("s") * 2 + lax.axis_index("c")
    pbase = wid * pw

    pltpu.sync_copy(offs.at[pl.ds(pbase, pw)], off_v)

    voff = (0, 9216, 96, 9312, 1, 9217, 97, 9313)

    def phase_a(cbase, st):
        idx_b, f_b, _ = st
        lanes = lax.iota(jnp.int32, L)
        c0 = lanes * 0
        for i in range(CH // L):
            fo = cbase + i * L + lanes
            yc = plsc.load_gather(off_v, [fo, c0])
            xc = plsc.load_gather(off_v, [fo, c0 + 1])
            zc = plsc.load_gather(off_v, [fo, c0 + 2])
            xi = xc.astype(jnp.int32)
            yi = yc.astype(jnp.int32)
            zi = zc.astype(jnp.int32)
            sl = pl.ds(i * L, L)
            f_b[0][sl] = zc - zi.astype(jnp.float32)
            f_b[1][sl] = xc - xi.astype(jnp.float32)
            f_b[2][sl] = yc - yi.astype(jnp.float32)
            v = yi * 9216 + xi * 96 + zi
            for c in range(8):
                idx_b[c][sl] = v + voff[c]

    def fire(st, sem):
        idx_b, _, r_b = st
        for c in range(8):
            pltpu.async_copy(table.at[idx_b[c]], r_b[c], sem)

    def drain(st, sem):
        idx_b, _, r_b = st
        for c in range(8):
            pltpu.make_async_copy(table.at[idx_b[c]], r_b[c], sem).wait()

    def blend(cbase, st):
        _, f_b, r_b = st
        lanes = lax.iota(jnp.int32, L)
        sel = lanes >> 3
        colid = lanes & 7
        for j in range(CH // 2):
            rvec = 2 * j + sel
            fz = plsc.load_gather(f_b[0], [rvec])
            fx = plsc.load_gather(f_b[1], [rvec])
            fy = plsc.load_gather(f_b[2], [rvec])
            i0 = plsc.load_gather(r_b[0], [rvec, colid])
            i1 = plsc.load_gather(r_b[1], [rvec, colid])
            i2 = plsc.load_gather(r_b[2], [rvec, colid])
            i3 = plsc.load_gather(r_b[3], [rvec, colid])
            i4 = plsc.load_gather(r_b[4], [rvec, colid])
            i5 = plsc.load_gather(r_b[5], [rvec, colid])
            i6 = plsc.load_gather(r_b[6], [rvec, colid])
            i7 = plsc.load_gather(r_b[7], [rvec, colid])
            a = i0 + fz * (i4 - i0)
            b = i1 + fz * (i5 - i1)
            c = i2 + fz * (i6 - i2)
            d = i3 + fz * (i7 - i3)
            e = a + fx * (c - a)
            f = b + fx * (d - b)
            plsc.store_scatter(out_v, [rvec, colid], e + fy * (f - e))
        pltpu.sync_copy(out_v, out.at[pl.ds(pbase + cbase, CH)])

    phase_a(0, sets[0])
    fire(sets[0], sem0)

    def pair(i, carry):
        cb0 = (2 * i) * CH
        cb1 = cb0 + CH
        cb2 = cb0 + 2 * CH
        phase_a(cb1, sets[1])
        fire(sets[1], sem1)
        drain(sets[0], sem0)
        blend(cb0, sets[0])

        @pl.when(cb2 < pw)
        def _():
            phase_a(cb2, sets[0])
            fire(sets[0], sem0)

        drain(sets[1], sem1)
        blend(cb1, sets[1])
        return carry

    lax.fori_loop(0, pw // CH // 2, pair, 0)


def kernel(im, offsets):
    B, H, W, D, C = im.shape
    N = offsets.shape[1]

    mesh = plsc.VectorSubcoreMesh(core_axis_name="c", subcore_axis_name="s")
    pw = N // NW
    scratch = [
        pltpu.VMEM((pw, 3), jnp.float32),
        pltpu.VMEM((CH, C), jnp.float32),
        pltpu.SemaphoreType.DMA,
        pltpu.SemaphoreType.DMA,
    ]
    for _ in range(2):
        scratch += [pltpu.VMEM((CH,), jnp.int32) for _ in range(8)]
        scratch += [pltpu.VMEM((CH,), jnp.float32) for _ in range(3)]
        scratch += [pltpu.VMEM((CH, C), jnp.float32) for _ in range(8)]

    run = pl.kernel(
        _body,
        out_type=jax.ShapeDtypeStruct((N, C), jnp.float32),
        mesh=mesh,
        scratch_types=scratch,
        compiler_params=pltpu.CompilerParams(
            needs_layout_passes=False, use_tc_tiling_on_sc=False),
    )

    outs = [run(im[b].reshape(H * W * D, C), offsets[b]) for b in range(B)]
    return jnp.stack(outs)


# speedup vs baseline: 1.6840x; 1.2370x over previous
import jax
import jax.numpy as jnp
from jax import lax
from jax.experimental import pallas as pl
from jax.experimental.pallas import tpu as pltpu
from jax.experimental.pallas import tpu_sc as plsc

NW = 32
CH = 128
L = 16
NBUF = 19


def _body(table, offs3, out3, off_v, out_v, sem0, sem1, *bufs):
    npts = offs3.shape[0] * offs3.shape[1]
    nb = offs3.shape[0]

    sets = []
    for s in range(2):
        grp = bufs[s * NBUF:(s + 1) * NBUF]
        sets.append((grp[0:8], grp[8:11], grp[11:19]))

    pw = npts // NW
    wid = lax.axis_index("s") * 2 + lax.axis_index("c")
    pbase = wid * pw
    wpb = NW // nb
    b = wid // wpb
    prow = pbase - b * offs3.shape[1]
    vbase = b * 96 * 96 * 96

    pltpu.sync_copy(offs3.at[pl.ds(b, 1), pl.ds(prow, pw)], off_v)

    voff = (0, 9216, 96, 9312, 1, 9217, 97, 9313)

    def phase_a(cbase, st):
        idx_b, f_b, _ = st
        lanes = lax.iota(jnp.int32, L)
        c0 = lanes * 0
        for i in range(CH // L):
            fo = cbase + i * L + lanes
            yc = plsc.load_gather(off_v, [c0, fo, c0])
            xc = plsc.load_gather(off_v, [c0, fo, c0 + 1])
            zc = plsc.load_gather(off_v, [c0, fo, c0 + 2])
            xi = xc.astype(jnp.int32)
            yi = yc.astype(jnp.int32)
            zi = zc.astype(jnp.int32)
            sl = pl.ds(i * L, L)
            f_b[0][sl] = zc - zi.astype(jnp.float32)
            f_b[1][sl] = xc - xi.astype(jnp.float32)
            f_b[2][sl] = yc - yi.astype(jnp.float32)
            v = vbase + yi * 9216 + xi * 96 + zi
            for c in range(8):
                idx_b[c][sl] = v + voff[c]

    def fire(st, sem):
        idx_b, _, r_b = st
        for c in range(8):
            pltpu.async_copy(table.at[idx_b[c]], r_b[c], sem)

    def drain(st, sem):
        idx_b, _, r_b = st
        for c in range(8):
            pltpu.make_async_copy(table.at[idx_b[c]], r_b[c], sem).wait()

    def blend(cbase, st):
        _, f_b, r_b = st
        lanes = lax.iota(jnp.int32, L)
        sel = lanes >> 3
        colid = lanes & 7
        for j in range(CH // 2):
            rvec = 2 * j + sel
            fz = plsc.load_gather(f_b[0], [rvec])
            fx = plsc.load_gather(f_b[1], [rvec])
            fy = plsc.load_gather(f_b[2], [rvec])
            i0 = plsc.load_gather(r_b[0], [rvec, colid])
            i1 = plsc.load_gather(r_b[1], [rvec, colid])
            i2 = plsc.load_gather(r_b[2], [rvec, colid])
            i3 = plsc.load_gather(r_b[3], [rvec, colid])
            i4 = plsc.load_gather(r_b[4], [rvec, colid])
            i5 = plsc.load_gather(r_b[5], [rvec, colid])
            i6 = plsc.load_gather(r_b[6], [rvec, colid])
            i7 = plsc.load_gather(r_b[7], [rvec, colid])
            a = i0 + fz * (i4 - i0)
            b = i1 + fz * (i5 - i1)
            c = i2 + fz * (i6 - i2)
            d = i3 + fz * (i7 - i3)
            e = a + fx * (c - a)
            f = b + fx * (d - b)
            plsc.store_scatter(out_v, [rvec, colid], e + fy * (f - e))
        pltpu.sync_copy(out_v, out3.at[pl.ds(pbase + cbase, CH)])

    phase_a(0, sets[0])
    fire(sets[0], sem0)

    def pair(i, carry):
        cb0 = (2 * i) * CH
        cb1 = cb0 + CH
        cb2 = cb0 + 2 * CH
        phase_a(cb1, sets[1])
        fire(sets[1], sem1)
        drain(sets[0], sem0)
        blend(cb0, sets[0])

        @pl.when(cb2 < pw)
        def _():
            phase_a(cb2, sets[0])
            fire(sets[0], sem0)

        drain(sets[1], sem1)
        blend(cb1, sets[1])
        return carry

    lax.fori_loop(0, pw // CH // 2, pair, 0)


def kernel(im, offsets):
    B, H, W, D, C = im.shape
    N = offsets.shape[1]

    mesh = plsc.VectorSubcoreMesh(core_axis_name="c", subcore_axis_name="s")
    pw = B * N // NW
    scratch = [
        pltpu.VMEM((1, pw, 3), jnp.float32),
        pltpu.VMEM((CH, C), jnp.float32),
        pltpu.SemaphoreType.DMA,
        pltpu.SemaphoreType.DMA,
    ]
    for _ in range(2):
        scratch += [pltpu.VMEM((CH,), jnp.int32) for _ in range(8)]
        scratch += [pltpu.VMEM((CH,), jnp.float32) for _ in range(3)]
        scratch += [pltpu.VMEM((CH, C), jnp.float32) for _ in range(8)]

    run = pl.kernel(
        _body,
        out_type=jax.ShapeDtypeStruct((B * N, C), jnp.float32),
        mesh=mesh,
        scratch_types=scratch,
        compiler_params=pltpu.CompilerParams(
            needs_layout_passes=False, use_tc_tiling_on_sc=False),
    )

    return run(im.reshape(B * H * W * D, C), offsets).reshape(B, N, C)


# speedup vs baseline: 1.7626x; 1.0467x over previous
import jax
import jax.numpy as jnp
from jax import lax
from jax.experimental import pallas as pl
from jax.experimental.pallas import tpu as pltpu
from jax.experimental.pallas import tpu_sc as plsc

NW = 32
CH = 128
L = 16
NBUF = 19


def _body(table, offs, out, off_v, out_v, sem0, sem1, *bufs):
    sets = []
    for s in range(2):
        grp = bufs[s * NBUF:(s + 1) * NBUF]
        sets.append((grp[0:8], grp[8:11], grp[11:19]))

    pw = out.shape[0] // (8 * NW)
    nchunk = pw // CH
    n_per_batch = 96 * 96 * 96

    wid = lax.axis_index("s") * 2 + lax.axis_index("c")
    pbase = wid * pw
    vbase = (pbase // (out.shape[0] // (8 * 2))) * n_per_batch

    pltpu.sync_copy(offs.at[pl.ds(pbase * 3, pw * 3)], off_v)

    voff = (0, 9216, 96, 9312, 1, 9217, 97, 9313)

    def phase_a(cbase, st):
        idx_b, f_b, _ = st
        lanes = lax.iota(jnp.int32, L)
        for i in range(CH // L):
            fo = (cbase + i * L + lanes) * 3
            yc = plsc.load_gather(off_v, [fo])
            xc = plsc.load_gather(off_v, [fo + 1])
            zc = plsc.load_gather(off_v, [fo + 2])
            xi = xc.astype(jnp.int32)
            yi = yc.astype(jnp.int32)
            zi = zc.astype(jnp.int32)
            sl = pl.ds(i * L, L)
            f_b[0][sl] = zc - zi.astype(jnp.float32)
            f_b[1][sl] = xc - xi.astype(jnp.float32)
            f_b[2][sl] = yc - yi.astype(jnp.float32)
            v = vbase + yi * 9216 + xi * 96 + zi
            for c in range(8):
                idx_b[c][sl] = v + voff[c]

    def fire(st, sem):
        idx_b, _, r_b = st
        for c in range(8):
            pltpu.async_copy(table.at[idx_b[c]], r_b[c], sem)

    def drain(st, sem):
        idx_b, _, r_b = st
        for c in range(8):
            pltpu.make_async_copy(table.at[idx_b[c]], r_b[c], sem).wait()

    def blend(cbase, st):
        _, f_b, r_b = st
        lanes = lax.iota(jnp.int32, L)
        sel = lanes >> 3
        colid = lanes & 7
        for j in range(CH // 2):
            rvec = 2 * j + sel
            fz = plsc.load_gather(f_b[0], [rvec])
            fx = plsc.load_gather(f_b[1], [rvec])
            fy = plsc.load_gather(f_b[2], [rvec])
            i0 = plsc.load_gather(r_b[0], [rvec, colid])
            i1 = plsc.load_gather(r_b[1], [rvec, colid])
            i2 = plsc.load_gather(r_b[2], [rvec, colid])
            i3 = plsc.load_gather(r_b[3], [rvec, colid])
            i4 = plsc.load_gather(r_b[4], [rvec, colid])
            i5 = plsc.load_gather(r_b[5], [rvec, colid])
            i6 = plsc.load_gather(r_b[6], [rvec, colid])
            i7 = plsc.load_gather(r_b[7], [rvec, colid])
            a = i0 + fz * (i4 - i0)
            b = i1 + fz * (i5 - i1)
            c = i2 + fz * (i6 - i2)
            d = i3 + fz * (i7 - i3)
            e = a + fx * (c - a)
            f = b + fx * (d - b)
            out_v[pl.ds(j * L, L)] = e + fy * (f - e)
        pltpu.sync_copy(out_v, out.at[pl.ds((pbase + cbase) * 8, CH * 8)])

    phase_a(0, sets[0])
    fire(sets[0], sem0)

    def pair(i, carry):
        cb0 = (2 * i) * CH
        cb1 = cb0 + CH
        cb2 = cb0 + 2 * CH
        phase_a(cb1, sets[1])
        fire(sets[1], sem1)
        drain(sets[0], sem0)
        blend(cb0, sets[0])

        @pl.when(cb2 < pw)
        def _():
            phase_a(cb2, sets[0])
            fire(sets[0], sem0)

        drain(sets[1], sem1)
        blend(cb1, sets[1])
        return carry

    lax.fori_loop(0, nchunk // 2, pair, 0)


def kernel(im, offsets):
    B, H, W, D, C = im.shape
    N = offsets.shape[1]
    table = im.reshape(B * H * W * D, C)
    offs = offsets.reshape(B * N * 3)

    mesh = plsc.VectorSubcoreMesh(core_axis_name="c", subcore_axis_name="s")
    pw = B * N // NW
    scratch = [
        pltpu.VMEM((pw * 3,), jnp.float32),
        pltpu.VMEM((CH * 8,), jnp.float32),
        pltpu.SemaphoreType.DMA,
        pltpu.SemaphoreType.DMA,
    ]
    for _ in range(2):
        scratch += [pltpu.VMEM((CH,), jnp.int32) for _ in range(8)]
        scratch += [pltpu.VMEM((CH,), jnp.float32) for _ in range(3)]
        scratch += [pltpu.VMEM((CH, C), jnp.float32) for _ in range(8)]

    run = pl.kernel(
        _body,
        out_type=jax.ShapeDtypeStruct((B * N * C,), jnp.float32),
        mesh=mesh,
        scratch_types=scratch,
        compiler_params=pltpu.CompilerParams(
            needs_layout_passes=False, use_tc_tiling_on_sc=False),
    )
    return run(table, offs).reshape(B, N, C)
